# baseline (speedup 1.0000x reference)
import functools

import jax
import jax.numpy as jnp
from jax import lax
from jax.experimental import pallas as pl
from jax.experimental.pallas import tpu as pltpu
from jax.experimental.pallas import tpu_sc as plsc

_NUM_FIELDS = 26
_FIELD_DIM = 100000
_EMBED_DIM = 16
_VOCAB = _NUM_FIELDS * _FIELD_DIM
_LANES = 16
_BB = 256
_NQ = 16384 // _BB
_PAIRS = _NUM_FIELDS * _NQ
_NBUF = 4
_LEAD = 2


def kernel(x, table):
    batch, num_fields = x.shape
    assert num_fields == _NUM_FIELDS and batch == 16384
    x1d = x.T.reshape(-1)
    t1d = table.T.reshape(-1)

    info = plsc.get_sparse_core_info()
    nw = info.num_cores * info.num_subcores
    ppw = _PAIRS // nw
    assert ppw * nw == _PAIRS and ppw % _NBUF == 0

    mesh = plsc.VectorSubcoreMesh(core_axis_name="c", subcore_axis_name="s")

    @functools.partial(
        pl.kernel,
        mesh=mesh,
        out_type=jax.ShapeDtypeStruct((_NUM_FIELDS, 2, _NQ, 2048), jnp.float32),
        scratch_types=[
            pltpu.VMEM((ppw * _BB,), jnp.int32),
            pltpu.VMEM((_NBUF, 4096), jnp.int32),
            pltpu.VMEM((_NBUF, 4096), jnp.float32),
            [pltpu.SemaphoreType.DMA] * _NBUF,
            [pltpu.SemaphoreType.DMA] * _NBUF,
        ],
        compiler_params=pltpu.CompilerParams(
            use_tc_tiling_on_sc=False, needs_layout_passes=False
        ),
    )
    def run(x_hbm, t_hbm, out_hbm, xr_v, idx_v, tiles_v, gsem, osem):
        wid = lax.axis_index("s") * info.num_cores + lax.axis_index("c")
        t0 = wid * ppw

        pltpu.sync_copy(x_hbm.at[pl.ds(t0 * _BB, ppw * _BB)], xr_v)

        def add_off(i, _):
            off = ((t0 + i) // _NQ) * _FIELD_DIM
            for j in range(_BB // _LANES):
                sl = pl.ds(i * _BB + j * _LANES, _LANES)
                xr_v[sl] = xr_v[sl] + off
            return 0

        lax.fori_loop(0, ppw, add_off, 0)

        def build_idx(i, b):
            for eb in range(2):
                for bsub in range(2):
                    for ei in range(8):
                        plane = (eb * 8 + ei) * _VOCAB
                        for c in range(8):
                            src = pl.ds(i * _BB + bsub * 128 + c * _LANES, _LANES)
                            dst = eb * 2048 + bsub * 1024 + ei * 128 + c * _LANES
                            idx_v[b, pl.ds(dst, _LANES)] = xr_v[src] + plane

        def start_gather(i, b):
            build_idx(i, b)
            pltpu.make_async_copy(
                t_hbm.at[idx_v.at[b]], tiles_v.at[b], gsem[b]
            ).start()

        def wait_gather(b):
            pltpu.make_async_copy(
                t_hbm.at[idx_v.at[b]], tiles_v.at[b], gsem[b]
            ).wait()

        def start_out(i, b):
            t = t0 + i
            f = t // _NQ
            q = t % _NQ
            for eb in range(2):
                pltpu.make_async_copy(
                    tiles_v.at[b, pl.ds(eb * 2048, 2048)],
                    out_hbm.at[f, eb, q],
                    osem[b],
                ).start()

        def wait_out(b):
            for eb in range(2):
                pltpu.make_async_copy(
                    tiles_v.at[b, pl.ds(eb * 2048, 2048)],
                    out_hbm.at[0, 0, 0],
                    osem[b],
                ).wait()

        for j in range(_LEAD):
            start_gather(j, j)

        def outer(m, _):
            for b in range(_NBUF):
                k = _NBUF * m + b
                wait_gather(b)
                start_out(k, b)
                bj = (b + _LEAD) % _NBUF

                @pl.when(k >= _NBUF - _LEAD)
                def _():
                    wait_out(bj)

                @pl.when(k + _LEAD < ppw)
                def _():
                    start_gather(k + _LEAD, bj)

            return 0

        lax.fori_loop(0, ppw // _NBUF, outer, 0)
        for b in range(_NBUF - _LEAD):
            wait_out((b + _LEAD) % _NBUF)

    out5 = run(x1d, t1d)
    return (
        out5.reshape(_NUM_FIELDS, 2, 128, 8, 128)
        .transpose(2, 4, 0, 1, 3)
        .reshape(batch, _NUM_FIELDS, _EMBED_DIM)
    )


# speedup vs baseline: 2.7340x; 2.7340x over previous
import functools

import jax
import jax.numpy as jnp
from jax import lax
from jax.experimental import pallas as pl
from jax.experimental.pallas import tpu as pltpu
from jax.experimental.pallas import tpu_sc as plsc

_NUM_FIELDS = 26
_FIELD_DIM = 100000
_EMBED_DIM = 16
_VOCAB = _NUM_FIELDS * _FIELD_DIM
_LANES = 16
_BB = 256
_NQ = 16384 // _BB
_PAIRS = _NUM_FIELDS * _NQ
_NBUF = 4


def kernel(x, table):
    batch, num_fields = x.shape
    assert num_fields == _NUM_FIELDS and batch == 16384
    x1d = x.T.reshape(-1)
    tp = jnp.pad(table, ((0, 0), (0, 112))).reshape(8 * _VOCAB, _EMBED_DIM)

    info = plsc.get_sparse_core_info()
    nw = info.num_cores * info.num_subcores
    ppw = _PAIRS // nw
    assert ppw * nw == _PAIRS and ppw % _NBUF == 0

    mesh = plsc.VectorSubcoreMesh(core_axis_name="c", subcore_axis_name="s")

    @functools.partial(
        pl.kernel,
        mesh=mesh,
        out_type=jax.ShapeDtypeStruct((_NUM_FIELDS, 2, _NQ, 2048), jnp.float32),
        scratch_types=[
            pltpu.VMEM((ppw * _BB,), jnp.int32),
            pltpu.VMEM((_NBUF, _BB, _EMBED_DIM), jnp.float32),
            pltpu.VMEM((_NBUF, 2, 2048), jnp.float32),
            [pltpu.SemaphoreType.DMA] * _NBUF,
            [pltpu.SemaphoreType.DMA] * _NBUF,
        ],
        compiler_params=pltpu.CompilerParams(
            use_tc_tiling_on_sc=False, needs_layout_passes=False
        ),
    )
    def run(x_hbm, table_hbm, out_hbm, idx_v, rows_v, tiles_v, gsem, osem):
        wid = lax.axis_index("s") * info.num_cores + lax.axis_index("c")
        t0 = wid * ppw

        pltpu.sync_copy(x_hbm.at[pl.ds(t0 * _BB, ppw * _BB)], idx_v)

        def add_off(i, _):
            off = ((t0 + i) // _NQ) * _FIELD_DIM
            for j in range(_BB // _LANES):
                sl = pl.ds(i * _BB + j * _LANES, _LANES)
                idx_v[sl] = (idx_v[sl] + off) * 8
            return 0

        lax.fori_loop(0, ppw, add_off, 0)

        def start_gather(i, b):
            pltpu.make_async_copy(
                table_hbm.at[idx_v.at[pl.ds(i * _BB, _BB)]],
                rows_v.at[b],
                gsem[b],
            ).start()

        def wait_gather(b):
            pltpu.make_async_copy(
                table_hbm.at[idx_v.at[pl.ds(0, _BB)]], rows_v.at[b], gsem[b]
            ).wait()

        def start_out(i, b):
            t = t0 + i
            f = t // _NQ
            q = t % _NQ
            for eb in range(2):
                pltpu.make_async_copy(
                    tiles_v.at[b, eb], out_hbm.at[f, eb, q], osem[b]
                ).start()

        def wait_out(b):
            for eb in range(2):
                pltpu.make_async_copy(
                    tiles_v.at[b, eb], out_hbm.at[0, 0, 0], osem[b]
                ).wait()

        lanes = lax.iota(jnp.int32, _LANES)

        def shuffle(b):
            for eb in range(2):
                for bsub in range(2):
                    for ei in range(8):
                        col = jnp.full((_LANES,), eb * 8 + ei, jnp.int32)
                        for c in range(8):
                            row = bsub * 128 + c * _LANES + lanes
                            v = plsc.load_gather(rows_v.at[b], [row, col])
                            dst = bsub * 1024 + ei * 128 + c * _LANES
                            tiles_v[b, eb, pl.ds(dst, _LANES)] = v

        for b in range(_NBUF):
            start_gather(b, b)

        def outer(m, _):
            for b in range(_NBUF):
                i = _NBUF * m + b
                wait_gather(b)

                @pl.when(m > 0)
                def _():
                    wait_out(b)

                shuffle(b)
                start_out(i, b)

                @pl.when(i + _NBUF < ppw)
                def _():
                    start_gather(i + _NBUF, b)

            return 0

        lax.fori_loop(0, ppw // _NBUF, outer, 0)
        for b in range(_NBUF):
            wait_out(b)

    out5 = run(x1d, tp)
    return (
        out5.reshape(_NUM_FIELDS, 2, 128, 8, 128)
        .transpose(2, 4, 0, 1, 3)
        .reshape(batch, _NUM_FIELDS, _EMBED_DIM)
    )


# speedup vs baseline: 2.7661x; 1.0117x over previous
import functools

import jax
import jax.numpy as jnp
from jax import lax
from jax.experimental import pallas as pl
from jax.experimental.pallas import tpu as pltpu
from jax.experimental.pallas import tpu_sc as plsc

_NUM_FIELDS = 26
_FIELD_DIM = 100000
_EMBED_DIM = 16
_VOCAB = _NUM_FIELDS * _FIELD_DIM
_LANES = 16
_BB = 256
_NQ = 16384 // _BB
_PAIRS = _NUM_FIELDS * _NQ
_NBUF = 4


def kernel(x, table):
    batch, num_fields = x.shape
    assert num_fields == _NUM_FIELDS and batch == 16384
    x1d = x.T.reshape(-1)

    info = plsc.get_sparse_core_info()
    nw = info.num_cores * info.num_subcores
    ppw = _PAIRS // nw
    assert ppw * nw == _PAIRS and ppw % _NBUF == 0

    mesh = plsc.VectorSubcoreMesh(core_axis_name="c", subcore_axis_name="s")

    @functools.partial(
        pl.kernel,
        mesh=mesh,
        out_type=jax.ShapeDtypeStruct((_NUM_FIELDS, 2, _NQ, 2048), jnp.float32),
        scratch_types=[
            pltpu.VMEM((ppw * _BB,), jnp.int32),
            pltpu.VMEM((_NBUF, _BB, _EMBED_DIM), jnp.float32),
            pltpu.VMEM((_NBUF, 2, 2048), jnp.float32),
            [pltpu.SemaphoreType.DMA] * _NBUF,
            [pltpu.SemaphoreType.DMA] * _NBUF,
        ],
        compiler_params=pltpu.CompilerParams(
            use_tc_tiling_on_sc=False, needs_layout_passes=False
        ),
    )
    def run(x_hbm, table_hbm, out_hbm, idx_v, rows_v, tiles_v, gsem, osem):
        wid = lax.axis_index("s") * info.num_cores + lax.axis_index("c")
        t0 = wid * ppw

        pltpu.sync_copy(x_hbm.at[pl.ds(t0 * _BB, ppw * _BB)], idx_v)

        def add_off(i, _):
            off = ((t0 + i) // _NQ) * _FIELD_DIM
            for j in range(_BB // _LANES):
                sl = pl.ds(i * _BB + j * _LANES, _LANES)
                idx_v[sl] = idx_v[sl] + off
            return 0

        lax.fori_loop(0, ppw, add_off, 0)

        def start_gather(i, b):
            pltpu.make_async_copy(
                table_hbm.at[idx_v.at[pl.ds(i * _BB, _BB)]],
                rows_v.at[b],
                gsem[b],
            ).start()

        def wait_gather(b):
            pltpu.make_async_copy(
                table_hbm.at[idx_v.at[pl.ds(0, _BB)]], rows_v.at[b], gsem[b]
            ).wait()

        def start_out(i, b):
            t = t0 + i
            f = t // _NQ
            q = t % _NQ
            for eb in range(2):
                pltpu.make_async_copy(
                    tiles_v.at[b, eb], out_hbm.at[f, eb, q], osem[b]
                ).start()

        def wait_out(b):
            for eb in range(2):
                pltpu.make_async_copy(
                    tiles_v.at[b, eb], out_hbm.at[0, 0, 0], osem[b]
                ).wait()

        lanes = lax.iota(jnp.int32, _LANES)

        def shuffle(b):
            for eb in range(2):
                for bsub in range(2):
                    for ei in range(8):
                        col = jnp.full((_LANES,), eb * 8 + ei, jnp.int32)
                        for c in range(8):
                            row = bsub * 128 + c * _LANES + lanes
                            v = plsc.load_gather(rows_v.at[b], [row, col])
                            dst = bsub * 1024 + ei * 128 + c * _LANES
                            tiles_v[b, eb, pl.ds(dst, _LANES)] = v

        for b in range(_NBUF):
            start_gather(b, b)

        def outer(m, _):
            for b in range(_NBUF):
                i = _NBUF * m + b
                wait_gather(b)

                @pl.when(m > 0)
                def _():
                    wait_out(b)

                shuffle(b)
                start_out(i, b)

                @pl.when(i + _NBUF < ppw)
                def _():
                    start_gather(i + _NBUF, b)

            return 0

        lax.fori_loop(0, ppw // _NBUF, outer, 0)
        for b in range(_NBUF):
            wait_out(b)

    out5 = run(x1d, table)
    return (
        out5.reshape(_NUM_FIELDS, 2, 128, 8, 128)
        .transpose(2, 4, 0, 1, 3)
        .reshape(batch, _NUM_FIELDS, _EMBED_DIM)
    )
